# Initial kernel scaffold; baseline (speedup 1.0000x reference)
import functools

import jax
import jax.numpy as jnp
from jax import lax
from jax.experimental import pallas as pl
from jax.experimental.pallas import tpu as pltpu
from jax.experimental.pallas import tpu_sc as plsc

N = 10000
E = 320000
D = 128

NC = 2
NS = 16
NW = NC * NS

CH = 128
NCH = E // CH
CPT = -(-NCH // NW)
RPT = N // NS

_f32 = jnp.float32


def _sc_aggregate(with_counts):
    mesh = plsc.VectorSubcoreMesh(core_axis_name="c", subcore_axis_name="s")

    out_type = [jax.ShapeDtypeStruct((NC * N, D), _f32)]
    if with_counts:
        out_type.append(jax.ShapeDtypeStruct((NC * N, 16), _f32))

    scratch_types = [
        pltpu.VMEM((CH,), jnp.int32),
        pltpu.VMEM((CH,), jnp.int32),
        pltpu.VMEM((CH, D), _f32),
        pltpu.VMEM((CH, 16), _f32),
        pltpu.VMEM((RPT // 5, D), _f32),
        pltpu.VMEM((RPT, 16), _f32),
        pltpu.VMEM_SHARED((N, D), _f32),
        pltpu.VMEM_SHARED((N, 16), _f32),
        pltpu.SemaphoreType.DMA,
    ]

    @functools.partial(pl.kernel, mesh=mesh, out_type=out_type,
                       scratch_types=scratch_types)
    def agg(x_hbm, src_hbm, dst_hbm, *refs):
        if with_counts:
            sums_out, cnt_out = refs[0], refs[1]
            scratch = refs[2:]
        else:
            sums_out = refs[0]
            cnt_out = None
            scratch = refs[1:]
        (srcv, dstv, rows, ones, zs, zc, sums_sh, cnt_sh, sem) = scratch

        cid = lax.axis_index("c")
        sid = lax.axis_index("s")
        wid = cid * NS + sid
        base = sid * RPT

        @pl.loop(0, RPT // 5)
        def _(i):
            @pl.loop(0, D, step=16)
            def _(j):
                zs[i, pl.ds(j, 16)] = jnp.zeros((16,), _f32)

        @pl.loop(0, RPT)
        def _(i):
            zc[i, :] = jnp.zeros((16,), _f32)

        if with_counts:
            @pl.loop(0, CH)
            def _(i):
                ones[i, :] = jnp.ones((16,), _f32)

        @pl.loop(0, 5)
        def _(k5):
            pltpu.sync_copy(zs, sums_sh.at[pl.ds(base + k5 * (RPT // 5),
                                                 RPT // 5)])
        pltpu.sync_copy(zc, cnt_sh.at[pl.ds(base, RPT)])
        plsc.subcore_barrier()

        @pl.loop(0, CPT)
        def _(it):
            c = wid + NW * it

            @pl.when(c < NCH)
            def _():
                off = pl.multiple_of(c * CH, CH)
                pltpu.sync_copy(src_hbm.at[pl.ds(off, CH)], srcv)
                pltpu.sync_copy(dst_hbm.at[pl.ds(off, CH)], dstv)
                pltpu.async_copy(x_hbm.at[srcv], rows, sem).wait()
                pltpu.sync_copy(rows, sums_sh.at[dstv], add=True)
                if with_counts:
                    pltpu.sync_copy(ones, cnt_sh.at[dstv], add=True)

        plsc.subcore_barrier()

        obase = cid * N + base
        pltpu.sync_copy(sums_sh.at[pl.ds(base, RPT)],
                        sums_out.at[pl.ds(obase, RPT)])
        if with_counts:
            pltpu.sync_copy(cnt_sh.at[pl.ds(base, RPT)],
                            cnt_out.at[pl.ds(obase, RPT)])

    return agg


_agg_first = _sc_aggregate(with_counts=True)
_agg_rest = _sc_aggregate(with_counts=False)


BN = 1250
GRID = N // BN


def _tc_body(s0, s1, c0, c1, x_ref, wl, wr, b, o_ref, *, pool):
    s = s0[...] + s1[...]
    deg = c0[...][:, 0:1] + c1[...][:, 0:1]
    mean = s * (1.0 / jnp.maximum(deg, 1.0))
    h = (jnp.dot(mean, wl[...], precision=lax.Precision.HIGHEST)
         + jnp.dot(x_ref[...], wr[...], precision=lax.Precision.HIGHEST)
         + b[...])
    h = jnp.maximum(h, 0.0)
    if pool:
        @pl.when(pl.program_id(0) == 0)
        def _():
            o_ref[...] = jnp.zeros_like(o_ref)
        o_ref[...] += jnp.sum(h, axis=0, keepdims=True)
    else:
        o_ref[...] = h


def _tc_layer(sums, counts, x, WlT, WrT, b2, pool):
    if pool:
        out_shape = jax.ShapeDtypeStruct((1, D), _f32)
        out_spec = pl.BlockSpec((1, D), lambda i: (0, 0))
    else:
        out_shape = jax.ShapeDtypeStruct((N, D), _f32)
        out_spec = pl.BlockSpec((BN, D), lambda i: (i, 0))
    return pl.pallas_call(
        functools.partial(_tc_body, pool=pool),
        grid=(GRID,),
        in_specs=[
            pl.BlockSpec((BN, D), lambda i: (i, 0)),
            pl.BlockSpec((BN, D), lambda i: (i + GRID, 0)),
            pl.BlockSpec((BN, 16), lambda i: (i, 0)),
            pl.BlockSpec((BN, 16), lambda i: (i + GRID, 0)),
            pl.BlockSpec((BN, D), lambda i: (i, 0)),
            pl.BlockSpec((D, D), lambda i: (0, 0)),
            pl.BlockSpec((D, D), lambda i: (0, 0)),
            pl.BlockSpec((1, D), lambda i: (0, 0)),
        ],
        out_specs=out_spec,
        out_shape=out_shape,
    )(sums, sums, counts, counts, x, WlT, WrT, b2)


def kernel(x, edge_index, Wl1, bl1, Wr1, Wl2, bl2, Wr2, Wl3, bl3, Wr3):
    src = edge_index[0].astype(jnp.int32)
    dst = edge_index[1].astype(jnp.int32)

    sums1, counts = _agg_first(x, src, dst)
    h1 = _tc_layer(sums1, counts, x, Wl1.T, Wr1.T, bl1.reshape(1, D), False)
    sums2 = _agg_rest(h1, src, dst)
    h2 = _tc_layer(sums2, counts, h1, Wl2.T, Wr2.T, bl2.reshape(1, D), False)
    sums3 = _agg_rest(h2, src, dst)
    return _tc_layer(sums3, counts, h2, Wl3.T, Wr3.T, bl3.reshape(1, D), True)


# speedup vs baseline: 13.2304x; 13.2304x over previous
import dataclasses
import functools

import jax
import jax.numpy as jnp
from jax import lax
from jax.experimental import pallas as pl
from jax.experimental.pallas import tpu as pltpu
from jax.experimental.pallas import tpu_sc as plsc

N = 10000
E = 320000
D = 128

NC = 2
NS = 16
NW = NC * NS

CH = 128
RT = 80
NCH = E // CH
NCHP = NW * RT
EP = NCHP * CH
NPAD = N + 16
ZB = 640
BLK = 8
NBLK = RT // BLK

_f32 = jnp.float32


@functools.cache
def _sc_aggregate(with_counts):
    mesh = plsc.VectorSubcoreMesh(core_axis_name="c", subcore_axis_name="s")

    out_type = [jax.ShapeDtypeStruct((NC * N, D), _f32)]
    if with_counts:
        out_type.append(jax.ShapeDtypeStruct((NW, N), _f32))

    scratch_types = [
        pltpu.VMEM((2, BLK * CH), jnp.int32),
        pltpu.VMEM((2, BLK, CH), jnp.int32),
        pltpu.VMEM((CH, D), _f32),
        pltpu.VMEM((CH, D), _f32),
        pltpu.VMEM_SHARED((NPAD, D), _f32),
        pltpu.SemaphoreType.DMA,
        pltpu.SemaphoreType.DMA,
        pltpu.SemaphoreType.DMA,
        pltpu.SemaphoreType.DMA,
        pltpu.SemaphoreType.DMA,
    ]
    if with_counts:
        scratch_types.insert(4, pltpu.VMEM((N,), _f32))

    cp = pltpu.CompilerParams()
    if "needs_layout_passes" in pltpu.CompilerParams.__dataclass_fields__:
        cp = dataclasses.replace(cp, needs_layout_passes=False)

    @functools.partial(pl.kernel, mesh=mesh, out_type=out_type,
                       scratch_types=scratch_types, compiler_params=cp)
    def agg(x_hbm, src_hbm, dst_hbm, z_hbm, *refs):
        if with_counts:
            sums_out, cnt_out = refs[0], refs[1]
            (srcb, dstb, rows0, rows1, cnt, sums_sh,
             semi0, semi1, sem0, sem1, semz) = refs[2:]
        else:
            sums_out = refs[0]
            cnt_out = None
            cnt = None
            (srcb, dstb, rows0, rows1, sums_sh,
             semi0, semi1, sem0, sem1, semz) = refs[1:]

        semi = (semi0, semi1)
        semr = (sem0, sem1)
        rows = (rows0, rows1)

        cid = lax.axis_index("c")
        sid = lax.axis_index("s")
        wid = cid * NS + sid
        base = sid * (N // NS)
        start = wid * RT

        def valid(it):
            return start + it < NCH

        pltpu.async_copy(z_hbm, sums_sh.at[pl.ds(base, ZB)], semz)

        def idxblk_copies(blk, b):
            off = pl.multiple_of((start + blk * BLK) * CH, CH)
            roff = pl.multiple_of(start + blk * BLK, BLK)
            return (
                pltpu.make_async_copy(src_hbm.at[pl.ds(off, BLK * CH)],
                                      srcb.at[b], semi[b]),
                pltpu.make_async_copy(dst_hbm.at[pl.ds(roff, BLK)],
                                      dstb.at[b], semi[b]),
            )

        def issue_idxblk(blk, b):
            for c in idxblk_copies(blk, b):
                c.start()

        def wait_idxblk(blk, b):
            for c in idxblk_copies(blk, b):
                c.wait()

        def gather_copy(bi, r, rb):
            idx = srcb.at[bi, pl.ds(r * CH, CH)]
            return pltpu.make_async_copy(x_hbm.at[idx], rows[rb], semr[rb])

        issue_idxblk(0, 0)
        issue_idxblk(1, 1)

        if with_counts:
            @pl.loop(0, N, step=16)
            def _(i):
                cnt[pl.ds(i, 16)] = jnp.zeros((16,), _f32)

        wait_idxblk(0, 0)
        gather_copy(0, 0, 0).start()

        pltpu.make_async_copy(z_hbm, sums_sh.at[pl.ds(base, ZB)],
                              semz).wait()
        plsc.subcore_barrier()

        def scatter(it, bi, r, rb):
            pltpu.sync_copy(rows[rb], sums_sh.at[dstb.at[bi, r]], add=True)
            if with_counts:
                for k in range(CH // 16):
                    idx = dstb[bi, r, pl.ds(k * 16, 16)]
                    plsc.addupdate_scatter(cnt, [idx], jnp.ones((16,), _f32))

        @pl.loop(0, NBLK // 2)
        def _(q):
            for half in (0, 1):
                blk = 2 * q + half
                for pr in range(BLK // 2):
                    r0 = 2 * pr
                    r1 = r0 + 1
                    r2 = r0 + 2
                    it0 = blk * BLK + r0
                    it1 = it0 + 1
                    it2 = it0 + 2

                    @pl.when(valid(it1))
                    def _():
                        gather_copy(half, r1, 1).start()

                    @pl.when(valid(it0))
                    def _():
                        gather_copy(half, r0, 0).wait()
                        scatter(it0, half, r0, 0)

                    if r2 < BLK:
                        @pl.when(valid(it2))
                        def _():
                            gather_copy(half, r2, 0).start()
                    else:
                        @pl.when((it2 < RT) & valid(it2))
                        def _():
                            wait_idxblk(blk + 1, 1 - half)
                            gather_copy(1 - half, 0, 0).start()

                    @pl.when(valid(it1))
                    def _():
                        gather_copy(half, r1, 1).wait()
                        scatter(it1, half, r1, 1)

                @pl.when((blk + 2 < NBLK) & valid((blk + 2) * BLK))
                def _():
                    issue_idxblk(blk + 2, half)

        plsc.subcore_barrier()

        RA = 632
        RLAST = N - RA * (NS - 1)

        @pl.when(sid < NS - 1)
        def _():
            pltpu.sync_copy(sums_sh.at[pl.ds(sid * RA, RA)],
                            sums_out.at[pl.ds(cid * N + sid * RA, RA)])

        @pl.when(sid == NS - 1)
        def _():
            pltpu.sync_copy(sums_sh.at[pl.ds(RA * (NS - 1), RLAST)],
                            sums_out.at[pl.ds(cid * N + RA * (NS - 1),
                                              RLAST)])

        if with_counts:
            pltpu.sync_copy(cnt, cnt_out.at[wid])

    return agg


def _rdeg_body(c_ref, o_ref):
    deg = jnp.sum(c_ref[...], axis=0, keepdims=True)
    o_ref[...] = 1.0 / jnp.maximum(deg, 1.0)


def _rdeg(counts):
    return pl.pallas_call(
        _rdeg_body,
        out_shape=jax.ShapeDtypeStruct((1, N), _f32),
    )(counts)


BN = 2000
GRID = N // BN


def _tc_body(s0, s1, rd, x_ref, wl, wr, b, o_ref, *, pool):
    mean = (s0[...] + s1[...]) * rd[...]
    h = (jnp.dot(mean, wl[...], precision=lax.Precision.HIGHEST)
         + jnp.dot(x_ref[...], wr[...], precision=lax.Precision.HIGHEST)
         + b[...])
    h = jnp.maximum(h, 0.0)
    if pool:
        @pl.when(pl.program_id(0) == 0)
        def _():
            o_ref[...] = jnp.zeros_like(o_ref)
        o_ref[...] += jnp.sum(h, axis=0, keepdims=True)
    else:
        o_ref[...] = h


def _tc_layer(sums, rdeg, x, WlT, WrT, b2, pool):
    if pool:
        out_shape = jax.ShapeDtypeStruct((1, D), _f32)
        out_spec = pl.BlockSpec((1, D), lambda i: (0, 0))
    else:
        out_shape = jax.ShapeDtypeStruct((N, D), _f32)
        out_spec = pl.BlockSpec((BN, D), lambda i: (i, 0))
    return pl.pallas_call(
        functools.partial(_tc_body, pool=pool),
        grid=(GRID,),
        in_specs=[
            pl.BlockSpec((BN, D), lambda i: (i, 0)),
            pl.BlockSpec((BN, D), lambda i: (i + GRID, 0)),
            pl.BlockSpec((BN, 1), lambda i: (i, 0)),
            pl.BlockSpec((BN, D), lambda i: (i, 0)),
            pl.BlockSpec((D, D), lambda i: (0, 0)),
            pl.BlockSpec((D, D), lambda i: (0, 0)),
            pl.BlockSpec((1, D), lambda i: (0, 0)),
        ],
        out_specs=out_spec,
        out_shape=out_shape,
    )(sums, sums, rdeg, x, WlT, WrT, b2)


def kernel(x, edge_index, Wl1, bl1, Wr1, Wl2, bl2, Wr2, Wl3, bl3, Wr3):
    pad = EP - E
    src = jnp.concatenate(
        [edge_index[0].astype(jnp.int32), jnp.zeros((pad,), jnp.int32)])
    dst = jnp.concatenate(
        [edge_index[1].astype(jnp.int32),
         jnp.full((pad,), N, jnp.int32)]).reshape(NCHP, CH)

    zblk = jnp.zeros((ZB, D), _f32)
    sums1, counts = _sc_aggregate(True)(x, src, dst, zblk)
    rdeg = _rdeg(counts).reshape(N, 1)
    h1 = _tc_layer(sums1, rdeg, x, Wl1.T, Wr1.T, bl1.reshape(1, D), False)
    sums2, = _sc_aggregate(False)(h1, src, dst, zblk)
    h2 = _tc_layer(sums2, rdeg, h1, Wl2.T, Wr2.T, bl2.reshape(1, D), False)
    sums3, = _sc_aggregate(False)(h2, src, dst, zblk)
    return _tc_layer(sums3, rdeg, h2, Wl3.T, Wr3.T, bl3.reshape(1, D), True)
